# Initial kernel scaffold; baseline (speedup 1.0000x reference)
import functools

import jax
import jax.numpy as jnp
from jax import lax
from jax.experimental import pallas as pl
from jax.experimental.pallas import tpu as pltpu
from jax.experimental.pallas import tpu_sc as plsc

N_NODES = 10000
N_EDGES = 320000
D = 128
N_ETYPES = 4

NC = 2
NS = 16
NW = NC * NS
L = 16

NPAD = 10240
EPAD = 327680
EPT = EPAD // NW
EROWS = EPAD // 128
CNT_VALID = 4 * N_NODES
CNT_DUMMY = CNT_VALID
CNT_ROWS = 40960
CPT = CNT_ROWS // NS

_MESH = plsc.VectorSubcoreMesh(core_axis_name="c", subcore_axis_name="s",
                               num_cores=NC, num_subcores=NS)

_f32 = jnp.float32
_i32 = jnp.int32


def _iota16():
    return lax.iota(_i32, L)


@functools.partial(
    pl.kernel,
    out_type=(
        jax.ShapeDtypeStruct((NPAD, D), _f32),
        jax.ShapeDtypeStruct((EPAD,), _f32),
    ),
    mesh=_MESH,
    scratch_types=dict(
        stage_i=pltpu.VMEM((16, 128), _i32),
        ones_v=pltpu.VMEM((128, L), _f32),
        cbuf=pltpu.VMEM((CPT, L), _f32),
        ccol=pltpu.VMEM((CPT,), _f32),
        rbuf=pltpu.VMEM((CNT_ROWS,), _f32),
        widx=pltpu.VMEM((EPT,), _i32),
        wbuf=pltpu.VMEM((EPT,), _f32),
        gidx_v=pltpu.VMEM((NPAD // NW,), _i32),
        gbuf=pltpu.VMEM((64, D), _f32),
        cnt16=pltpu.VMEM_SHARED((CNT_ROWS, L), _f32),
        cntc=pltpu.VMEM_SHARED((CNT_ROWS,), _f32),
        sem=pltpu.SemaphoreType.DMA,
    ),
)
def _sc_prep(xcat, gidx, cidx2d, cidx1d, h_out, w_out,
             stage_i, ones_v, cbuf, ccol, rbuf, widx, wbuf, gidx_v, gbuf,
             cnt16, cntc, sem):
    cid = lax.axis_index("c")
    sid = lax.axis_index("s")
    wid = sid * NC + cid

    def _z(r, _):
        cbuf[r] = jnp.zeros((L,), _f32)
        return _
    lax.fori_loop(0, CPT, _z, None)
    pltpu.sync_copy(cbuf, cnt16.at[pl.ds(sid * CPT, CPT)])

    def _o(r, _):
        ones_v[r] = jnp.ones((L,), _f32)
        return _
    lax.fori_loop(0, 128, _o, None)
    plsc.subcore_barrier()

    for g in range(10):
        pltpu.sync_copy(cidx2d.at[pl.ds(sid * 160 + g * 16, 16)], stage_i)
        for j in range(16):
            pltpu.sync_copy(ones_v, cnt16.at[stage_i.at[j]], add=True)
    plsc.subcore_barrier()

    pltpu.sync_copy(cnt16.at[pl.ds(sid * CPT, CPT)], cbuf)
    zero16 = jnp.zeros((L,), _i32)

    def _c(r, _):
        rows = r * L + _iota16()
        ccol[pl.ds(r * L, L)] = plsc.load_gather(cbuf, [rows, zero16])
        return _
    lax.fori_loop(0, CPT // L, _c, None)
    pltpu.sync_copy(ccol, cntc.at[pl.ds(sid * CPT, CPT)])
    plsc.subcore_barrier()

    pltpu.sync_copy(cntc, rbuf)

    def _r(r, _):
        v = rbuf[pl.ds(r * L, L)]
        rbuf[pl.ds(r * L, L)] = 1.0 / jnp.maximum(v, 1.0)
        return _
    lax.fori_loop(0, CNT_ROWS // L, _r, None)

    def _dz(r, _):
        rbuf[pl.ds(CNT_VALID + r * L, L)] = jnp.zeros((L,), _f32)
        return _
    lax.fori_loop(0, (CNT_ROWS - CNT_VALID) // L, _dz, None)

    pltpu.sync_copy(cidx1d.at[pl.ds(wid * EPT, EPT)], widx)

    def _w(k, _):
        iv = widx[pl.ds(k * L, L)]
        wbuf[pl.ds(k * L, L)] = plsc.load_gather(rbuf, [iv])
        return _
    lax.fori_loop(0, EPT // L, _w, None)
    pltpu.sync_copy(wbuf, w_out.at[pl.ds(wid * EPT, EPT)])

    npw = NPAD // NW
    pltpu.sync_copy(gidx.at[pl.ds(wid * npw, npw)], gidx_v)
    for c in range(npw // 64):
        pltpu.async_copy(xcat.at[gidx_v.at[pl.ds(c * 64, 64)]], gbuf, sem).wait()
        pltpu.sync_copy(gbuf, h_out.at[pl.ds(wid * npw + c * 64, 64)])


@functools.partial(
    pl.kernel,
    out_type=jax.ShapeDtypeStruct((NC, NPAD, D), _f32),
    mesh=_MESH,
    scratch_types=dict(
        src_v=pltpu.VMEM((EPT,), _i32),
        w_v=pltpu.VMEM((EPT,), _f32),
        dstage=pltpu.VMEM((8, 128), _i32),
        buf=pltpu.VMEM((128, D), _f32),
        acc=pltpu.VMEM_SHARED((NPAD, D), _f32),
        sem=pltpu.SemaphoreType.DMA,
    ),
)
def _sc_agg(y, sidx, dst2d, w, pout, src_v, w_v, dstage, buf, acc, sem):
    cid = lax.axis_index("c")
    sid = lax.axis_index("s")
    wid = sid * NC + cid
    rpt = NPAD // NS

    def _z(r, _):
        buf[r] = jnp.zeros((D,), _f32)
        return _
    lax.fori_loop(0, 128, _z, None)
    for i in range(rpt // 128):
        pltpu.sync_copy(buf, acc.at[pl.ds(sid * rpt + i * 128, 128)])
    plsc.subcore_barrier()

    pltpu.sync_copy(sidx.at[pl.ds(wid * EPT, EPT)], src_v)
    pltpu.sync_copy(w.at[pl.ds(wid * EPT, EPT)], w_v)

    for g in range(10):
        pltpu.sync_copy(dst2d.at[pl.ds(wid * 80 + g * 8, 8)], dstage)
        for j in range(8):
            b = g * 8 + j
            pltpu.async_copy(y.at[src_v.at[pl.ds(b * 128, 128)]], buf, sem).wait()

            def _s(e, _, boff=b * 128):
                ws = w_v[boff + e]
                for c in range(D // L):
                    sl = pl.ds(c * L, L)
                    buf[e, sl] = buf[e, sl] * ws
                return _
            lax.fori_loop(0, 128, _s, None)
            pltpu.sync_copy(buf, acc.at[dstage.at[j]], add=True)
    plsc.subcore_barrier()

    pltpu.sync_copy(acc.at[pl.ds(sid * rpt, rpt)],
                    pout.at[cid, pl.ds(sid * rpt, rpt)])


def _dotT(a, b):
    return lax.dot_general(a, b, (((1,), (1,)), ((), ())),
                           preferred_element_type=_f32)


def _rel_root(hb, rw_ref, rtw_ref, rtb_ref, m, y_ref, root_ref):
    for t in range(N_ETYPES):
        y_ref[t] = _dotT(hb, rw_ref[t])
    r0 = _dotT(hb, rtw_ref[0]) + rtb_ref[0][None, :]
    r1 = _dotT(hb, rtw_ref[1]) + rtb_ref[1][None, :]
    root_ref[...] = m * r0 + (1.0 - m) * r1


def _tc_first_body(h_ref, rw_ref, rtw_ref, rtb_ref, ntm_ref, y_ref, root_ref):
    _rel_root(h_ref[...], rw_ref, rtw_ref, rtb_ref, ntm_ref[...], y_ref, root_ref)


def _tc_mid_body(p_ref, root0_ref, rw_ref, rtw_ref, rtb_ref, ntm_ref,
                 y_ref, root_ref):
    h1 = jnp.maximum(p_ref[0] + p_ref[1] + root0_ref[...], 0.0)
    _rel_root(h1, rw_ref, rtw_ref, rtb_ref, ntm_ref[...], y_ref, root_ref)


def _tc_fin_body(p_ref, root_ref, o_ref):
    s = p_ref[0] + p_ref[1] + root_ref[...]
    z = s - jnp.max(s, axis=1, keepdims=True)
    o_ref[...] = z - jnp.log(jnp.sum(jnp.exp(z), axis=1, keepdims=True))


_BLK = 1024
_GRID = NPAD // _BLK

_w_specs = [
    pl.BlockSpec((N_ETYPES, D, D), lambda i: (0, 0, 0)),
    pl.BlockSpec((2, D, D), lambda i: (0, 0, 0)),
    pl.BlockSpec((2, D), lambda i: (0, 0)),
    pl.BlockSpec((_BLK, D), lambda i: (i, 0)),
]
_y_root_out = (
    jax.ShapeDtypeStruct((N_ETYPES, NPAD, D), _f32),
    jax.ShapeDtypeStruct((NPAD, D), _f32),
)
_y_root_specs = (
    pl.BlockSpec((N_ETYPES, _BLK, D), lambda i: (0, i, 0)),
    pl.BlockSpec((_BLK, D), lambda i: (i, 0)),
)

_tc_first = pl.pallas_call(
    _tc_first_body,
    grid=(_GRID,),
    in_specs=[pl.BlockSpec((_BLK, D), lambda i: (i, 0))] + _w_specs,
    out_specs=_y_root_specs,
    out_shape=_y_root_out,
)

_tc_mid = pl.pallas_call(
    _tc_mid_body,
    grid=(_GRID,),
    in_specs=[pl.BlockSpec((NC, _BLK, D), lambda i: (0, i, 0)),
              pl.BlockSpec((_BLK, D), lambda i: (i, 0))] + _w_specs,
    out_specs=_y_root_specs,
    out_shape=_y_root_out,
)

_FBLK = 1000
_tc_fin = pl.pallas_call(
    _tc_fin_body,
    grid=(N_NODES // _FBLK,),
    in_specs=[pl.BlockSpec((NC, _FBLK, D), lambda i: (0, i, 0)),
              pl.BlockSpec((_FBLK, D), lambda i: (i, 0))],
    out_specs=pl.BlockSpec((_FBLK, D), lambda i: (i, 0)),
    out_shape=jax.ShapeDtypeStruct((N_NODES, D), _f32),
)


def kernel(x0, x1, edge_index, edge_type, node_type, local_node_idx,
           rel_w0, root_w0, root_b0, rel_w1, root_w1, root_b1):
    xcat = jnp.concatenate([x0, x1], axis=0)
    gidx = node_type * x0.shape[0] + local_node_idx
    gidx = jnp.pad(gidx, (0, NPAD - N_NODES)).astype(_i32)

    src = edge_index[0]
    dst = edge_index[1]
    et = edge_type
    epad = EPAD - N_EDGES
    cidx = jnp.pad(dst * N_ETYPES + et, (0, epad),
                   constant_values=CNT_DUMMY).astype(_i32)
    sidx = jnp.pad(et * NPAD + src, (0, epad)).astype(_i32)
    dstp = jnp.pad(dst, (0, epad)).astype(_i32)
    cidx2d = cidx.reshape(EROWS, 128)
    dst2d = dstp.reshape(EROWS, 128)
    ntm = jnp.broadcast_to(
        (node_type == 0).astype(_f32)[:, None], (N_NODES, D))
    ntm = jnp.pad(ntm, ((0, NPAD - N_NODES), (0, 0)))

    h, w = _sc_prep(xcat, gidx, cidx2d, cidx)

    y0, root0 = _tc_first(h, rel_w0, root_w0, root_b0, ntm)
    p0 = _sc_agg(y0.reshape(N_ETYPES * NPAD, D), sidx, dst2d, w)

    y1, root1 = _tc_mid(p0, root0, rel_w1, root_w1, root_b1, ntm)
    p1 = _sc_agg(y1.reshape(N_ETYPES * NPAD, D), sidx, dst2d, w)

    return _tc_fin(p1, root1)


# speedup vs baseline: 5.6091x; 5.6091x over previous
import functools

import jax
import jax.numpy as jnp
from jax import lax
from jax.experimental import pallas as pl
from jax.experimental.pallas import tpu as pltpu
from jax.experimental.pallas import tpu_sc as plsc

N_NODES = 10000
N_EDGES = 320000
D = 128
N_ETYPES = 4

NC = 2
NS = 16
NW = NC * NS
L = 16

NPAD = 10240
EPAD = 327680
EPT = EPAD // NW
ERPT = EPT // 128
EROWS = EPAD // 128
CNT_VALID = 4 * N_NODES
CNT_DUMMY = 40960
CROWS = 384
CRPT = CROWS // NS

_MESH = plsc.VectorSubcoreMesh(core_axis_name="c", subcore_axis_name="s",
                               num_cores=NC, num_subcores=NS)
_SC_PARAMS = pltpu.CompilerParams(needs_layout_passes=False)

_f32 = jnp.float32
_i32 = jnp.int32


def _iota16():
    return lax.iota(_i32, L)


@functools.partial(
    pl.kernel,
    out_type=jax.ShapeDtypeStruct((NC, CROWS, 128), _f32),
    mesh=_MESH,
    compiler_params=_SC_PARAMS,
    scratch_types=dict(
        cloc=pltpu.VMEM((CROWS, 128), _f32),
        widx=pltpu.VMEM((ERPT, 128), _i32),
        ident=pltpu.VMEM((3, 128), _i32),
        zbuf=pltpu.VMEM((CRPT, 128), _f32),
        cntS=pltpu.VMEM_SHARED((CROWS, 128), _f32),
    ),
)
def _sc_cnt(cidx2d, cnt_out, cloc, widx, ident, zbuf, cntS):
    cid = lax.axis_index("c")
    sid = lax.axis_index("s")
    wid = sid * NC + cid

    def _z(r, _):
        for q in range(8):
            cloc[r, pl.ds(q * L, L)] = jnp.zeros((L,), _f32)
        return _
    lax.fori_loop(0, CROWS, _z, None)
    for r in range(CRPT):
        for q in range(8):
            zbuf[r, pl.ds(q * L, L)] = jnp.zeros((L,), _f32)
    pltpu.sync_copy(zbuf, cntS.at[pl.ds(sid * CRPT, CRPT)])
    for r3 in range(3):
        for q in range(8):
            ident[r3, pl.ds(q * L, L)] = _iota16() + (r3 * 128 + q * L)

    pltpu.sync_copy(cidx2d.at[pl.ds(wid * ERPT, ERPT)], widx)
    ones16 = jnp.ones((L,), _f32)

    def _h(r, _):
        for q in range(8):
            iv = widx[r, pl.ds(q * L, L)]
            plsc.addupdate_scatter(cloc, [iv // 128, iv % 128], ones16)
        return _
    lax.fori_loop(0, ERPT, _h, None)
    plsc.subcore_barrier()

    for r3 in range(3):
        pltpu.sync_copy(cloc.at[pl.ds(r3 * 128, 128)],
                        cntS.at[ident.at[r3]], add=True)
    plsc.subcore_barrier()

    pltpu.sync_copy(cntS.at[pl.ds(sid * CRPT, CRPT)],
                    cnt_out.at[cid, pl.ds(sid * CRPT, CRPT)])


@functools.partial(
    pl.kernel,
    out_type=(
        jax.ShapeDtypeStruct((NPAD, D), _f32),
        jax.ShapeDtypeStruct((EROWS, 128), _f32),
    ),
    mesh=_MESH,
    compiler_params=_SC_PARAMS,
    scratch_types=dict(
        rbuf=pltpu.VMEM((CROWS, 128), _f32),
        gbuf=pltpu.VMEM((128, D), _f32),
        widx=pltpu.VMEM((ERPT, 128), _i32),
        wbuf=pltpu.VMEM((ERPT, 128), _f32),
        gidx_v=pltpu.VMEM((4, 128), _i32),
        sem=pltpu.SemaphoreType.DMA,
    ),
)
def _sc_prep(xcat, gidx, cidx2d, cnt, h_out, w_out,
             rbuf, gbuf, widx, wbuf, gidx_v, sem):
    cid = lax.axis_index("c")
    sid = lax.axis_index("s")
    wid = sid * NC + cid

    pltpu.sync_copy(cnt.at[0], rbuf)
    for c3 in range(3):
        pltpu.sync_copy(cnt.at[1, pl.ds(c3 * 128, 128)], gbuf)

        def _a(r, _, base=c3 * 128):
            for q in range(8):
                sl = pl.ds(q * L, L)
                rbuf[base + r, sl] = rbuf[base + r, sl] + gbuf[r, sl]
            return _
        lax.fori_loop(0, 128, _a, None)

    def _r(r, _):
        for q in range(8):
            sl = pl.ds(q * L, L)
            rbuf[r, sl] = 1.0 / jnp.maximum(rbuf[r, sl], 1.0)
        return _
    lax.fori_loop(0, CROWS, _r, None)

    pltpu.sync_copy(cidx2d.at[pl.ds(wid * ERPT, ERPT)], widx)

    def _w(r, _):
        for q in range(8):
            iv = widx[r, pl.ds(q * L, L)]
            wbuf[r, pl.ds(q * L, L)] = plsc.load_gather(
                rbuf, [iv // 128, iv % 128])
        return _
    lax.fori_loop(0, ERPT, _w, None)

    @pl.when(wid == NW - 1)
    def _ztail():
        for r in range(20, ERPT):
            for q in range(8):
                wbuf[r, pl.ds(q * L, L)] = jnp.zeros((L,), _f32)

    pltpu.sync_copy(wbuf, w_out.at[pl.ds(wid * ERPT, ERPT)])

    @pl.when(wid < 20)
    def _hgather():
        pltpu.sync_copy(gidx.at[pl.ds(wid * 4, 4)], gidx_v)
        for r in range(4):
            pltpu.async_copy(xcat.at[gidx_v.at[r]], gbuf, sem).wait()
            pltpu.sync_copy(gbuf, h_out.at[pl.ds((wid * 4 + r) * 128, 128)])


@functools.partial(
    pl.kernel,
    out_type=jax.ShapeDtypeStruct((NC, NPAD, D), _f32),
    mesh=_MESH,
    compiler_params=_SC_PARAMS,
    scratch_types=dict(
        src_v=pltpu.VMEM((ERPT, 128), _i32),
        w_v=pltpu.VMEM((ERPT, 128), _f32),
        dstage=pltpu.VMEM((8, 128), _i32),
        buf=pltpu.VMEM((128, D), _f32),
        acc=pltpu.VMEM_SHARED((NPAD, D), _f32),
        sem=pltpu.SemaphoreType.DMA,
    ),
)
def _sc_agg(y, sidx, dst2d, w, pout, src_v, w_v, dstage, buf, acc, sem):
    cid = lax.axis_index("c")
    sid = lax.axis_index("s")
    wid = sid * NC + cid
    rpt = NPAD // NS

    def _z(r, _):
        for c in range(D // L):
            buf[r, pl.ds(c * L, L)] = jnp.zeros((L,), _f32)
        return _
    lax.fori_loop(0, 128, _z, None)
    for i in range(rpt // 128):
        pltpu.sync_copy(buf, acc.at[pl.ds(sid * rpt + i * 128, 128)])
    plsc.subcore_barrier()

    pltpu.sync_copy(sidx.at[pl.ds(wid * ERPT, ERPT)], src_v)
    pltpu.sync_copy(w.at[pl.ds(wid * ERPT, ERPT)], w_v)

    for g in range(10):
        pltpu.sync_copy(dst2d.at[pl.ds(wid * ERPT + g * 8, 8)], dstage)
        for j in range(8):
            b = g * 8 + j
            pltpu.async_copy(y.at[src_v.at[b]], buf, sem).wait()

            def _s(e, _, brow=b):
                wv = plsc.load_gather(
                    w_v, [jnp.full((L,), brow, _i32), jnp.full((L,), e, _i32)])
                for c in range(D // L):
                    sl = pl.ds(c * L, L)
                    buf[e, sl] = buf[e, sl] * wv
                return _
            lax.fori_loop(0, 128, _s, None)
            pltpu.sync_copy(buf, acc.at[dstage.at[j]], add=True)
    plsc.subcore_barrier()

    pltpu.sync_copy(acc.at[pl.ds(sid * rpt, rpt)],
                    pout.at[cid, pl.ds(sid * rpt, rpt)])


def _dotT(a, b):
    return lax.dot_general(a, b, (((1,), (1,)), ((), ())),
                           preferred_element_type=_f32)


def _rel_root(hb, rw_ref, rtw_ref, rtb_ref, m, y_ref, root_ref):
    for t in range(N_ETYPES):
        y_ref[t] = _dotT(hb, rw_ref[t])
    r0 = _dotT(hb, rtw_ref[0]) + rtb_ref[0][None, :]
    r1 = _dotT(hb, rtw_ref[1]) + rtb_ref[1][None, :]
    root_ref[...] = m * r0 + (1.0 - m) * r1


def _tc_first_body(h_ref, rw_ref, rtw_ref, rtb_ref, ntm_ref, y_ref, root_ref):
    _rel_root(h_ref[...], rw_ref, rtw_ref, rtb_ref, ntm_ref[...], y_ref, root_ref)


def _tc_mid_body(p_ref, root0_ref, rw_ref, rtw_ref, rtb_ref, ntm_ref,
                 y_ref, root_ref):
    h1 = jnp.maximum(p_ref[0] + p_ref[1] + root0_ref[...], 0.0)
    _rel_root(h1, rw_ref, rtw_ref, rtb_ref, ntm_ref[...], y_ref, root_ref)


def _tc_fin_body(p_ref, root_ref, o_ref):
    s = p_ref[0] + p_ref[1] + root_ref[...]
    z = s - jnp.max(s, axis=1, keepdims=True)
    o_ref[...] = z - jnp.log(jnp.sum(jnp.exp(z), axis=1, keepdims=True))


_BLK = 1024
_GRID = NPAD // _BLK

_w_specs = [
    pl.BlockSpec((N_ETYPES, D, D), lambda i: (0, 0, 0)),
    pl.BlockSpec((2, D, D), lambda i: (0, 0, 0)),
    pl.BlockSpec((2, D), lambda i: (0, 0)),
    pl.BlockSpec((_BLK, D), lambda i: (i, 0)),
]
_y_root_out = (
    jax.ShapeDtypeStruct((N_ETYPES, NPAD, D), _f32),
    jax.ShapeDtypeStruct((NPAD, D), _f32),
)
_y_root_specs = (
    pl.BlockSpec((N_ETYPES, _BLK, D), lambda i: (0, i, 0)),
    pl.BlockSpec((_BLK, D), lambda i: (i, 0)),
)

_tc_first = pl.pallas_call(
    _tc_first_body,
    grid=(_GRID,),
    in_specs=[pl.BlockSpec((_BLK, D), lambda i: (i, 0))] + _w_specs,
    out_specs=_y_root_specs,
    out_shape=_y_root_out,
)

_tc_mid = pl.pallas_call(
    _tc_mid_body,
    grid=(_GRID,),
    in_specs=[pl.BlockSpec((NC, _BLK, D), lambda i: (0, i, 0)),
              pl.BlockSpec((_BLK, D), lambda i: (i, 0))] + _w_specs,
    out_specs=_y_root_specs,
    out_shape=_y_root_out,
)

_FBLK = 1000
_tc_fin = pl.pallas_call(
    _tc_fin_body,
    grid=(N_NODES // _FBLK,),
    in_specs=[pl.BlockSpec((NC, _FBLK, D), lambda i: (0, i, 0)),
              pl.BlockSpec((_FBLK, D), lambda i: (i, 0))],
    out_specs=pl.BlockSpec((_FBLK, D), lambda i: (i, 0)),
    out_shape=jax.ShapeDtypeStruct((N_NODES, D), _f32),
)


def kernel(x0, x1, edge_index, edge_type, node_type, local_node_idx,
           rel_w0, root_w0, root_b0, rel_w1, root_w1, root_b1):
    xcat = jnp.concatenate([x0, x1], axis=0)
    gidx = node_type * x0.shape[0] + local_node_idx
    gidx2d = jnp.pad(gidx, (0, NPAD - N_NODES)).astype(_i32).reshape(80, 128)

    src = edge_index[0]
    dst = edge_index[1]
    et = edge_type
    epad = EPAD - N_EDGES
    cidx = jnp.pad(dst * N_ETYPES + et, (0, epad),
                   constant_values=CNT_DUMMY).astype(_i32)
    sidx = jnp.pad(et * NPAD + src, (0, epad)).astype(_i32).reshape(EROWS, 128)
    dstp = jnp.pad(dst, (0, epad)).astype(_i32)
    cidx2d = cidx.reshape(EROWS, 128)
    dst2d = dstp.reshape(EROWS, 128)
    ntm = jnp.broadcast_to(
        (node_type == 0).astype(_f32)[:, None], (N_NODES, D))
    ntm = jnp.pad(ntm, ((0, NPAD - N_NODES), (0, 0)))

    cnt = _sc_cnt(cidx2d)
    h, w = _sc_prep(xcat, gidx2d, cidx2d, cnt)

    y0, root0 = _tc_first(h, rel_w0, root_w0, root_b0, ntm)
    p0 = _sc_agg(y0.reshape(N_ETYPES * NPAD, D), sidx, dst2d, w)

    y1, root1 = _tc_mid(p0, root0, rel_w1, root_w1, root_b1, ntm)
    p1 = _sc_agg(y1.reshape(N_ETYPES * NPAD, D), sidx, dst2d, w)

    return _tc_fin(p1, root1)
